# baseline (speedup 1.0000x reference)
import functools

import jax
import jax.numpy as jnp
from jax import lax
from jax.experimental import pallas as pl
from jax.experimental.pallas import tpu as pltpu
from jax.experimental.pallas import tpu_sc as plsc

N_NODES = 10000
N_EDGES = 320000
D = 128
DH = D // 2
NC = 2
NS = 16
NW = NC * NS
EPT = N_EDGES // NW
CHUNK = 128
NCHUNK = 160
EPT1 = NCHUNK * CHUNK
NPAD = 10240
RPT = NPAD // NS

_mesh = plsc.VectorSubcoreMesh(
    core_axis_name="c", subcore_axis_name="s", num_cores=NC, num_subcores=NS
)


@functools.partial(
    pl.kernel,
    out_type=jax.ShapeDtypeStruct((NC, NPAD, DH), jnp.float32),
    mesh=_mesh,
    scratch_types=[
        pltpu.VMEM((NCHUNK, CHUNK), jnp.int32),
        pltpu.VMEM((NCHUNK, CHUNK), jnp.int32),
        pltpu.VMEM((4, CHUNK, DH), jnp.float32),
        pltpu.VMEM((32, DH), jnp.float32),
        pltpu.VMEM_SHARED((NPAD, DH), jnp.float32),
        pltpu.SemaphoreType.DMA,
        pltpu.SemaphoreType.DMA,
        pltpu.SemaphoreType.DMA,
        pltpu.SemaphoreType.DMA,
    ],
    compiler_params=pltpu.CompilerParams(use_tc_tiling_on_sc=False),
)
def _segment_sum_sc(xr_hbm, col_hbm, row_hbm, out_hbm,
                    col_v, row_v, bufs, zbuf, acc, sem0, sem1, sem2, sem3):
    c = lax.axis_index("c")
    s = lax.axis_index("s")

    zero16 = jnp.zeros((16,), jnp.float32)
    for i in range(32):
        for k in range(DH // 16):
            zbuf[i, pl.ds(k * 16, 16)] = zero16
    base_row = s * RPT
    for k in range(RPT // 32):
        pltpu.sync_copy(zbuf, acc.at[pl.ds(base_row + k * 32, 32)])
    plsc.subcore_barrier()

    pltpu.sync_copy(col_hbm.at[s], col_v)
    pltpu.sync_copy(row_hbm.at[s], row_v)

    @pl.when(c == 1)
    def _():
        def addc(i, carry):
            for k in range(CHUNK // 16):
                col_v[i, pl.ds(k * 16, 16)] = col_v[i, pl.ds(k * 16, 16)] + 1
            return carry

        lax.fori_loop(0, NCHUNK, addc, 0)

    x_hbm = xr_hbm
    sems = (sem0, sem1, sem2, sem3)

    for b in range(3):
        pltpu.async_copy(x_hbm.at[col_v.at[b]], bufs.at[b], sems[b])

    def body(g, carry):
        for b in range(4):
            j = g * 4 + b

            @pl.when(j + 3 < NCHUNK)
            def _():
                pltpu.async_copy(
                    x_hbm.at[col_v.at[j + 3]], bufs.at[(b + 3) % 4], sems[(b + 3) % 4]
                )

            pltpu.make_async_copy(x_hbm.at[col_v.at[j]], bufs.at[b], sems[b]).wait()
            pltpu.sync_copy(bufs.at[b], acc.at[row_v.at[j]], add=True)
        return carry

    lax.fori_loop(0, NCHUNK // 4, body, 0)
    plsc.subcore_barrier()

    for k in range(RPT // 128):
        r0 = base_row + k * 128
        pltpu.sync_copy(acc.at[pl.ds(r0, 128)], out_hbm.at[c].at[pl.ds(r0, 128)])


def _dense_tc_body(x_ref, p_ref, ws_ref, wn_ref, bc_ref, w8_ref, b8_ref, out_ref):
    wn = wn_ref[...]
    h = jnp.maximum(
        jnp.dot(x_ref[...], ws_ref[...], preferred_element_type=jnp.float32)
        + jnp.dot(p_ref[0], wn[:DH, :], preferred_element_type=jnp.float32)
        + jnp.dot(p_ref[1], wn[DH:, :], preferred_element_type=jnp.float32)
        + bc_ref[...],
        0.0,
    )
    out_ref[...] = (
        jnp.dot(h, w8_ref[...], preferred_element_type=jnp.float32) + b8_ref[...]
    )


_BN = 2000


def _dense_tc(x, partials, W_self, W_neigh, bc, w8, b8):
    return pl.pallas_call(
        _dense_tc_body,
        grid=(N_NODES // _BN,),
        in_specs=[
            pl.BlockSpec((_BN, D), lambda i: (i, 0)),
            pl.BlockSpec((NC, _BN, DH), lambda i: (0, i, 0)),
            pl.BlockSpec((D, D), lambda i: (0, 0)),
            pl.BlockSpec((D, D), lambda i: (0, 0)),
            pl.BlockSpec((1, D), lambda i: (0, 0)),
            pl.BlockSpec((D, 8), lambda i: (0, 0)),
            pl.BlockSpec((1, 8), lambda i: (0, 0)),
        ],
        out_specs=pl.BlockSpec((_BN, 8), lambda i: (i, 0)),
        out_shape=jax.ShapeDtypeStruct((N_NODES, 8), jnp.float32),
    )(x, partials, W_self, W_neigh, bc, w8, b8)


@functools.partial(
    pl.kernel,
    out_type=jax.ShapeDtypeStruct((2, N_EDGES), jnp.float32),
    mesh=_mesh,
    scratch_types=[
        pltpu.VMEM((N_NODES * 4,), jnp.float32),
        pltpu.VMEM((EPT,), jnp.int32),
        pltpu.VMEM((EPT,), jnp.int32),
        pltpu.VMEM((EPT,), jnp.float32),
        pltpu.VMEM((EPT,), jnp.float32),
    ],
    compiler_params=pltpu.CompilerParams(
        needs_layout_passes=False, use_tc_tiling_on_sc=False
    ),
)
def _edge_logits_sc(tab_hbm, row_hbm, col_hbm, out_hbm, tab_v, row_v, col_v, o0_v, o1_v):
    c = lax.axis_index("c")
    s = lax.axis_index("s")
    wid = c * NS + s
    pltpu.sync_copy(tab_hbm, tab_v)
    pltpu.sync_copy(row_hbm.at[pl.ds(wid * EPT, EPT)], row_v)
    pltpu.sync_copy(col_hbm.at[pl.ds(wid * EPT, EPT)], col_v)

    def body(i, carry):
        r16 = row_v[pl.ds(i * 16, 16)] * 4
        c16 = col_v[pl.ds(i * 16, 16)] * 4
        o0_v[pl.ds(i * 16, 16)] = (
            plsc.load_gather(tab_v, [r16]) + plsc.load_gather(tab_v, [c16 + 2])
        )
        o1_v[pl.ds(i * 16, 16)] = (
            plsc.load_gather(tab_v, [r16 + 1]) + plsc.load_gather(tab_v, [c16 + 3])
        )
        return carry

    lax.fori_loop(0, EPT // 16, body, 0)
    pltpu.sync_copy(o0_v, out_hbm.at[0].at[pl.ds(wid * EPT, EPT)])
    pltpu.sync_copy(o1_v, out_hbm.at[1].at[pl.ds(wid * EPT, EPT)])


def kernel(x, edge_index, W_self, W_neigh, b_core, W_node, b_node, W_edge, b_edge):
    row = edge_index[0].astype(jnp.int32)
    col = edge_index[1].astype(jnp.int32)
    pad = NS * EPT1 - N_EDGES
    col3 = jnp.concatenate([col * 2, jnp.zeros((pad,), jnp.int32)])
    row3 = jnp.concatenate([row, jnp.full((pad,), N_NODES, jnp.int32)])
    partials = _segment_sum_sc(
        x.reshape(2 * N_NODES, DH),
        col3.reshape(NS, NCHUNK, CHUNK),
        row3.reshape(NS, NCHUNK, CHUNK),
    )
    w8 = (
        jnp.zeros((D, 8), jnp.float32)
        .at[:, 0:2].set(W_node)
        .at[:, 2:4].set(W_edge[:D])
        .at[:, 4:6].set(W_edge[D:])
    )
    b8 = (
        jnp.zeros((8,), jnp.float32)
        .at[0:2].set(b_node)
        .at[2:4].set(b_edge)
    )
    out8 = _dense_tc(
        x, partials, W_self, W_neigh, b_core.reshape(1, D), w8, b8.reshape(1, 8)
    )
    node_logits = out8[:, 0:2]
    tab = out8[:, 2:6].reshape(-1)
    edge_logits = _edge_logits_sc(tab, row, col).T
    return (node_logits, edge_logits)


# speedup vs baseline: 2.3012x; 2.3012x over previous
import functools

import jax
import jax.numpy as jnp
from jax import lax
from jax.experimental import pallas as pl
from jax.experimental.pallas import tpu as pltpu
from jax.experimental.pallas import tpu_sc as plsc

N_NODES = 10000
N_EDGES = 320000
D = 128
DH = D // 2
NC = 2
NS = 16
NW = NC * NS
EPT = N_EDGES // NW
EPT1 = N_EDGES // NS
CHUNK = 250
NCHUNK = EPT1 // CHUNK
RING = 3
NPAD = 10240
RPT = NPAD // NS

_mesh = plsc.VectorSubcoreMesh(
    core_axis_name="c", subcore_axis_name="s", num_cores=NC, num_subcores=NS
)


@functools.partial(
    pl.kernel,
    out_type=jax.ShapeDtypeStruct((NC, NPAD, DH), jnp.float32),
    mesh=_mesh,
    scratch_types=[
        pltpu.VMEM((NCHUNK, CHUNK), jnp.int32),
        pltpu.VMEM((NCHUNK, CHUNK), jnp.int32),
        pltpu.VMEM((RING, CHUNK, DH), jnp.float32),
        pltpu.VMEM((16, DH), jnp.float32),
        pltpu.VMEM_SHARED((NPAD, DH), jnp.float32),
        pltpu.SemaphoreType.DMA,
        pltpu.SemaphoreType.DMA,
        pltpu.SemaphoreType.DMA,
    ],
    compiler_params=pltpu.CompilerParams(use_tc_tiling_on_sc=False),
)
def _segment_sum_sc(xs_hbm, col_hbm, row_hbm, out_hbm,
                    col_v, row_v, bufs, zbuf, acc, sem0, sem1, sem2):
    c = lax.axis_index("c")
    s = lax.axis_index("s")

    zero16 = jnp.zeros((16,), jnp.float32)
    for i in range(16):
        for k in range(DH // 16):
            zbuf[i, pl.ds(k * 16, 16)] = zero16
    base_row = s * RPT
    for k in range(RPT // 16):
        pltpu.sync_copy(zbuf, acc.at[pl.ds(base_row + k * 16, 16)])
    plsc.subcore_barrier()

    pltpu.sync_copy(col_hbm.at[s], col_v)
    pltpu.sync_copy(row_hbm.at[s], row_v)

    x_hbm = xs_hbm.at[c]
    sems = (sem0, sem1, sem2)

    for b in range(RING - 1):
        pltpu.async_copy(x_hbm.at[col_v.at[b]], bufs.at[b], sems[b])

    def body(g, carry):
        for b in range(RING):
            j = g * RING + b

            @pl.when(j + RING - 1 < NCHUNK)
            def _():
                pltpu.async_copy(
                    x_hbm.at[col_v.at[j + RING - 1]],
                    bufs.at[(b + RING - 1) % RING],
                    sems[(b + RING - 1) % RING],
                )

            @pl.when(j < NCHUNK)
            def _():
                pltpu.make_async_copy(
                    x_hbm.at[col_v.at[j]], bufs.at[b], sems[b]
                ).wait()
                pltpu.sync_copy(bufs.at[b], acc.at[row_v.at[j]], add=True)

        return carry

    lax.fori_loop(0, (NCHUNK + RING - 1) // RING, body, 0)
    plsc.subcore_barrier()

    for k in range(RPT // 128):
        r0 = base_row + k * 128
        pltpu.sync_copy(acc.at[pl.ds(r0, 128)], out_hbm.at[c].at[pl.ds(r0, 128)])


def _dense_tc_body(x_ref, p_ref, ws_ref, wn_ref, bc_ref, w8_ref, b8_ref, out_ref):
    wn = wn_ref[...]
    h = jnp.maximum(
        jnp.dot(x_ref[...], ws_ref[...], preferred_element_type=jnp.float32)
        + jnp.dot(p_ref[0], wn[:DH, :], preferred_element_type=jnp.float32)
        + jnp.dot(p_ref[1], wn[DH:, :], preferred_element_type=jnp.float32)
        + bc_ref[...],
        0.0,
    )
    out_ref[...] = (
        jnp.dot(h, w8_ref[...], preferred_element_type=jnp.float32) + b8_ref[...]
    )


_BN = 2000


def _dense_tc(x, partials, W_self, W_neigh, bc, w8, b8):
    return pl.pallas_call(
        _dense_tc_body,
        grid=(N_NODES // _BN,),
        in_specs=[
            pl.BlockSpec((_BN, D), lambda i: (i, 0)),
            pl.BlockSpec((NC, _BN, DH), lambda i: (0, i, 0)),
            pl.BlockSpec((D, D), lambda i: (0, 0)),
            pl.BlockSpec((D, D), lambda i: (0, 0)),
            pl.BlockSpec((1, D), lambda i: (0, 0)),
            pl.BlockSpec((D, 8), lambda i: (0, 0)),
            pl.BlockSpec((1, 8), lambda i: (0, 0)),
        ],
        out_specs=pl.BlockSpec((_BN, 8), lambda i: (i, 0)),
        out_shape=jax.ShapeDtypeStruct((N_NODES, 8), jnp.float32),
    )(x, partials, W_self, W_neigh, bc, w8, b8)


@functools.partial(
    pl.kernel,
    out_type=jax.ShapeDtypeStruct((2, N_EDGES), jnp.float32),
    mesh=_mesh,
    scratch_types=[
        pltpu.VMEM((N_NODES * 4,), jnp.float32),
        pltpu.VMEM((EPT,), jnp.int32),
        pltpu.VMEM((EPT,), jnp.int32),
        pltpu.VMEM((EPT,), jnp.float32),
        pltpu.VMEM((EPT,), jnp.float32),
    ],
    compiler_params=pltpu.CompilerParams(
        needs_layout_passes=False, use_tc_tiling_on_sc=False
    ),
)
def _edge_logits_sc(tab_hbm, row_hbm, col_hbm, out_hbm, tab_v, row_v, col_v, o0_v, o1_v):
    c = lax.axis_index("c")
    s = lax.axis_index("s")
    wid = c * NS + s
    pltpu.sync_copy(tab_hbm, tab_v)
    pltpu.sync_copy(row_hbm.at[pl.ds(wid * EPT, EPT)], row_v)
    pltpu.sync_copy(col_hbm.at[pl.ds(wid * EPT, EPT)], col_v)

    def body(i, carry):
        r16 = row_v[pl.ds(i * 16, 16)] * 4
        c16 = col_v[pl.ds(i * 16, 16)] * 4
        o0_v[pl.ds(i * 16, 16)] = (
            plsc.load_gather(tab_v, [r16]) + plsc.load_gather(tab_v, [c16 + 2])
        )
        o1_v[pl.ds(i * 16, 16)] = (
            plsc.load_gather(tab_v, [r16 + 1]) + plsc.load_gather(tab_v, [c16 + 3])
        )
        return carry

    lax.fori_loop(0, EPT // 16, body, 0)
    pltpu.sync_copy(o0_v, out_hbm.at[0].at[pl.ds(wid * EPT, EPT)])
    pltpu.sync_copy(o1_v, out_hbm.at[1].at[pl.ds(wid * EPT, EPT)])


def kernel(x, edge_index, W_self, W_neigh, b_core, W_node, b_node, W_edge, b_edge):
    row = edge_index[0].astype(jnp.int32)
    col = edge_index[1].astype(jnp.int32)
    xs = jnp.stack([x[:, :DH], x[:, DH:]])
    partials = _segment_sum_sc(
        xs, col.reshape(NS, NCHUNK, CHUNK), row.reshape(NS, NCHUNK, CHUNK)
    )
    w8 = (
        jnp.zeros((D, 8), jnp.float32)
        .at[:, 0:2].set(W_node)
        .at[:, 2:4].set(W_edge[:D])
        .at[:, 4:6].set(W_edge[D:])
    )
    b8 = (
        jnp.zeros((8,), jnp.float32)
        .at[0:2].set(b_node)
        .at[2:4].set(b_edge)
    )
    out8 = _dense_tc(
        x, partials, W_self, W_neigh, b_core.reshape(1, D), w8, b8.reshape(1, 8)
    )
    node_logits = out8[:, 0:2]
    tab = out8[:, 2:6].reshape(-1)
    edge_logits = _edge_logits_sc(tab, row, col).T
    return (node_logits, edge_logits)


# speedup vs baseline: 2.3106x; 1.0041x over previous
import functools

import jax
import jax.numpy as jnp
from jax import lax
from jax.experimental import pallas as pl
from jax.experimental.pallas import tpu as pltpu
from jax.experimental.pallas import tpu_sc as plsc

N_NODES = 10000
N_EDGES = 320000
D = 128
DH = D // 2
NC = 2
NS = 16
NW = NC * NS
EPT = N_EDGES // NW
EPT1 = N_EDGES // NS
CHUNK = 125
NCHUNK = EPT1 // CHUNK
RING = 4
NPAD = 10240
RPT = NPAD // NS

_mesh = plsc.VectorSubcoreMesh(
    core_axis_name="c", subcore_axis_name="s", num_cores=NC, num_subcores=NS
)


@functools.partial(
    pl.kernel,
    out_type=jax.ShapeDtypeStruct((NC, NPAD, DH), jnp.float32),
    mesh=_mesh,
    scratch_types=[
        pltpu.VMEM((NCHUNK, CHUNK), jnp.int32),
        pltpu.VMEM((NCHUNK, CHUNK), jnp.int32),
        pltpu.VMEM((RING, CHUNK, DH), jnp.float32),
        pltpu.VMEM((16, DH), jnp.float32),
        pltpu.VMEM_SHARED((NPAD, DH), jnp.float32),
        pltpu.SemaphoreType.DMA,
        pltpu.SemaphoreType.DMA,
        pltpu.SemaphoreType.DMA,
        pltpu.SemaphoreType.DMA,
    ],
    compiler_params=pltpu.CompilerParams(use_tc_tiling_on_sc=False),
)
def _segment_sum_sc(xs_hbm, col_hbm, row_hbm, out_hbm,
                    col_v, row_v, bufs, zbuf, acc, sem0, sem1, sem2, sem3):
    c = lax.axis_index("c")
    s = lax.axis_index("s")

    zero16 = jnp.zeros((16,), jnp.float32)
    for i in range(16):
        for k in range(DH // 16):
            zbuf[i, pl.ds(k * 16, 16)] = zero16
    base_row = s * RPT
    for k in range(RPT // 16):
        pltpu.sync_copy(zbuf, acc.at[pl.ds(base_row + k * 16, 16)])
    plsc.subcore_barrier()

    pltpu.sync_copy(col_hbm.at[s], col_v)
    pltpu.sync_copy(row_hbm.at[s], row_v)

    x_hbm = xs_hbm.at[c]
    sems = (sem0, sem1, sem2, sem3)

    for b in range(RING - 1):
        pltpu.async_copy(x_hbm.at[col_v.at[b]], bufs.at[b], sems[b])

    def body(g, carry):
        for b in range(RING):
            j = g * RING + b

            @pl.when(j + RING - 1 < NCHUNK)
            def _():
                pltpu.async_copy(
                    x_hbm.at[col_v.at[j + RING - 1]],
                    bufs.at[(b + RING - 1) % RING],
                    sems[(b + RING - 1) % RING],
                )

            @pl.when(j < NCHUNK)
            def _():
                pltpu.make_async_copy(
                    x_hbm.at[col_v.at[j]], bufs.at[b], sems[b]
                ).wait()
                pltpu.sync_copy(bufs.at[b], acc.at[row_v.at[j]], add=True)

        return carry

    lax.fori_loop(0, (NCHUNK + RING - 1) // RING, body, 0)
    plsc.subcore_barrier()

    for k in range(RPT // 128):
        r0 = base_row + k * 128
        pltpu.sync_copy(acc.at[pl.ds(r0, 128)], out_hbm.at[c].at[pl.ds(r0, 128)])


def _dense_tc_body(x_ref, p_ref, ws_ref, wn_ref, bc_ref, w8_ref, b8_ref, out_ref):
    wn = wn_ref[...]
    h = jnp.maximum(
        jnp.dot(x_ref[...], ws_ref[...], preferred_element_type=jnp.float32)
        + jnp.dot(p_ref[0], wn[:DH, :], preferred_element_type=jnp.float32)
        + jnp.dot(p_ref[1], wn[DH:, :], preferred_element_type=jnp.float32)
        + bc_ref[...],
        0.0,
    )
    out_ref[...] = (
        jnp.dot(h, w8_ref[...], preferred_element_type=jnp.float32) + b8_ref[...]
    )


_BN = 2000


def _dense_tc(x, partials, W_self, W_neigh, bc, w8, b8):
    return pl.pallas_call(
        _dense_tc_body,
        grid=(N_NODES // _BN,),
        in_specs=[
            pl.BlockSpec((_BN, D), lambda i: (i, 0)),
            pl.BlockSpec((NC, _BN, DH), lambda i: (0, i, 0)),
            pl.BlockSpec((D, D), lambda i: (0, 0)),
            pl.BlockSpec((D, D), lambda i: (0, 0)),
            pl.BlockSpec((1, D), lambda i: (0, 0)),
            pl.BlockSpec((D, 8), lambda i: (0, 0)),
            pl.BlockSpec((1, 8), lambda i: (0, 0)),
        ],
        out_specs=pl.BlockSpec((_BN, 8), lambda i: (i, 0)),
        out_shape=jax.ShapeDtypeStruct((N_NODES, 8), jnp.float32),
    )(x, partials, W_self, W_neigh, bc, w8, b8)


@functools.partial(
    pl.kernel,
    out_type=jax.ShapeDtypeStruct((2, N_EDGES), jnp.float32),
    mesh=_mesh,
    scratch_types=[
        pltpu.VMEM((N_NODES * 4,), jnp.float32),
        pltpu.VMEM((EPT,), jnp.int32),
        pltpu.VMEM((EPT,), jnp.int32),
        pltpu.VMEM((EPT,), jnp.float32),
        pltpu.VMEM((EPT,), jnp.float32),
    ],
    compiler_params=pltpu.CompilerParams(
        needs_layout_passes=False, use_tc_tiling_on_sc=False
    ),
)
def _edge_logits_sc(tab_hbm, row_hbm, col_hbm, out_hbm, tab_v, row_v, col_v, o0_v, o1_v):
    c = lax.axis_index("c")
    s = lax.axis_index("s")
    wid = c * NS + s
    pltpu.sync_copy(tab_hbm, tab_v)
    pltpu.sync_copy(row_hbm.at[pl.ds(wid * EPT, EPT)], row_v)
    pltpu.sync_copy(col_hbm.at[pl.ds(wid * EPT, EPT)], col_v)

    def body(i, carry):
        r16 = row_v[pl.ds(i * 16, 16)] * 4
        c16 = col_v[pl.ds(i * 16, 16)] * 4
        o0_v[pl.ds(i * 16, 16)] = (
            plsc.load_gather(tab_v, [r16]) + plsc.load_gather(tab_v, [c16 + 2])
        )
        o1_v[pl.ds(i * 16, 16)] = (
            plsc.load_gather(tab_v, [r16 + 1]) + plsc.load_gather(tab_v, [c16 + 3])
        )
        return carry

    lax.fori_loop(0, EPT // 16, body, 0)
    pltpu.sync_copy(o0_v, out_hbm.at[0].at[pl.ds(wid * EPT, EPT)])
    pltpu.sync_copy(o1_v, out_hbm.at[1].at[pl.ds(wid * EPT, EPT)])


def kernel(x, edge_index, W_self, W_neigh, b_core, W_node, b_node, W_edge, b_edge):
    row = edge_index[0].astype(jnp.int32)
    col = edge_index[1].astype(jnp.int32)
    xs = jnp.stack([x[:, :DH], x[:, DH:]])
    partials = _segment_sum_sc(
        xs, col.reshape(NS, NCHUNK, CHUNK), row.reshape(NS, NCHUNK, CHUNK)
    )
    w8 = (
        jnp.zeros((D, 8), jnp.float32)
        .at[:, 0:2].set(W_node)
        .at[:, 2:4].set(W_edge[:D])
        .at[:, 4:6].set(W_edge[D:])
    )
    b8 = (
        jnp.zeros((8,), jnp.float32)
        .at[0:2].set(b_node)
        .at[2:4].set(b_edge)
    )
    out8 = _dense_tc(
        x, partials, W_self, W_neigh, b_core.reshape(1, D), w8, b8.reshape(1, 8)
    )
    node_logits = out8[:, 0:2]
    tab = out8[:, 2:6].reshape(-1)
    edge_logits = _edge_logits_sc(tab, row, col).T
    return (node_logits, edge_logits)


# speedup vs baseline: 2.4083x; 1.0423x over previous
import functools

import jax
import jax.numpy as jnp
from jax import lax
from jax.experimental import pallas as pl
from jax.experimental.pallas import tpu as pltpu
from jax.experimental.pallas import tpu_sc as plsc

N_NODES = 10000
N_EDGES = 320000
D = 128
DH = D // 2
NC = 2
NS = 16
NW = NC * NS
EPT = N_EDGES // NW
EPT1 = N_EDGES // NS
CHUNK = 125
NCHUNK = EPT1 // CHUNK
RING = 4
NPAD = 10240
RPT = NPAD // NS

_mesh = plsc.VectorSubcoreMesh(
    core_axis_name="c", subcore_axis_name="s", num_cores=NC, num_subcores=NS
)


@functools.partial(
    pl.kernel,
    out_type=jax.ShapeDtypeStruct((NC, NPAD, DH), jnp.float32),
    mesh=_mesh,
    scratch_types=[
        pltpu.VMEM((NCHUNK, CHUNK), jnp.int32),
        pltpu.VMEM((NCHUNK, CHUNK), jnp.int32),
        pltpu.VMEM((RING, CHUNK, DH), jnp.float32),
        pltpu.VMEM((16, DH), jnp.float32),
        pltpu.VMEM_SHARED((NPAD, DH), jnp.float32),
        pltpu.SemaphoreType.DMA,
        pltpu.SemaphoreType.DMA,
        pltpu.SemaphoreType.DMA,
        pltpu.SemaphoreType.DMA,
    ],
    compiler_params=pltpu.CompilerParams(use_tc_tiling_on_sc=False),
)
def _segment_sum_sc(xs_hbm, col_hbm, row_hbm, out_hbm,
                    col_v, row_v, bufs, zbuf, acc, sem0, sem1, sem2, sem3):
    c = lax.axis_index("c")
    s = lax.axis_index("s")

    zero16 = jnp.zeros((16,), jnp.float32)
    for i in range(16):
        for k in range(DH // 16):
            zbuf[i, pl.ds(k * 16, 16)] = zero16
    base_row = s * RPT
    for k in range(RPT // 16):
        pltpu.sync_copy(zbuf, acc.at[pl.ds(base_row + k * 16, 16)])
    plsc.subcore_barrier()

    pltpu.sync_copy(col_hbm.at[s], col_v)
    pltpu.sync_copy(row_hbm.at[s], row_v)

    x_hbm = xs_hbm.at[c]
    sems = (sem0, sem1, sem2, sem3)

    for b in range(RING - 1):
        pltpu.async_copy(x_hbm.at[col_v.at[b]], bufs.at[b], sems[b])

    def body(g, carry):
        for b in range(RING):
            j = g * RING + b

            @pl.when(j + RING - 1 < NCHUNK)
            def _():
                pltpu.async_copy(
                    x_hbm.at[col_v.at[j + RING - 1]],
                    bufs.at[(b + RING - 1) % RING],
                    sems[(b + RING - 1) % RING],
                )

            @pl.when(j < NCHUNK)
            def _():
                pltpu.make_async_copy(
                    x_hbm.at[col_v.at[j]], bufs.at[b], sems[b]
                ).wait()
                pltpu.sync_copy(bufs.at[b], acc.at[row_v.at[j]], add=True)

        return carry

    lax.fori_loop(0, (NCHUNK + RING - 1) // RING, body, 0)
    plsc.subcore_barrier()

    for k in range(RPT // 128):
        r0 = base_row + k * 128
        pltpu.sync_copy(acc.at[pl.ds(r0, 128)], out_hbm.at[c].at[pl.ds(r0, 128)])


def _dense_tc_body(x_ref, p_ref, ws_ref, wn_ref, bc_ref, w8_ref, b8_ref,
                   nl_ref, tab_ref):
    wn = wn_ref[...]
    h = jnp.maximum(
        jnp.dot(x_ref[...], ws_ref[...], preferred_element_type=jnp.float32)
        + jnp.dot(p_ref[0], wn[:DH, :], preferred_element_type=jnp.float32)
        + jnp.dot(p_ref[1], wn[DH:, :], preferred_element_type=jnp.float32)
        + bc_ref[...],
        0.0,
    )
    out8 = jnp.dot(h, w8_ref[...], preferred_element_type=jnp.float32) + b8_ref[...]
    nl_ref[...] = out8[:, 0:2].T
    tab_ref[...] = out8[:, 2:6].T


_BN = 1280


def _dense_tc(xp, partials, W_self, W_neigh, bc, w8, b8):
    return pl.pallas_call(
        _dense_tc_body,
        grid=(NPAD // _BN,),
        in_specs=[
            pl.BlockSpec((_BN, D), lambda i: (i, 0)),
            pl.BlockSpec((NC, _BN, DH), lambda i: (0, i, 0)),
            pl.BlockSpec((D, D), lambda i: (0, 0)),
            pl.BlockSpec((D, D), lambda i: (0, 0)),
            pl.BlockSpec((1, D), lambda i: (0, 0)),
            pl.BlockSpec((D, 8), lambda i: (0, 0)),
            pl.BlockSpec((1, 8), lambda i: (0, 0)),
        ],
        out_specs=[
            pl.BlockSpec((2, _BN), lambda i: (0, i)),
            pl.BlockSpec((4, _BN), lambda i: (0, i)),
        ],
        out_shape=[
            jax.ShapeDtypeStruct((2, NPAD), jnp.float32),
            jax.ShapeDtypeStruct((4, NPAD), jnp.float32),
        ],
    )(xp, partials, W_self, W_neigh, bc, w8, b8)


@functools.partial(
    pl.kernel,
    out_type=jax.ShapeDtypeStruct((2, N_EDGES), jnp.float32),
    mesh=_mesh,
    scratch_types=[
        pltpu.VMEM((NPAD * 4,), jnp.float32),
        pltpu.VMEM((EPT,), jnp.int32),
        pltpu.VMEM((EPT,), jnp.int32),
        pltpu.VMEM((EPT,), jnp.float32),
        pltpu.VMEM((EPT,), jnp.float32),
    ],
    compiler_params=pltpu.CompilerParams(
        needs_layout_passes=False, use_tc_tiling_on_sc=False
    ),
)
def _edge_logits_sc(tab_hbm, row_hbm, col_hbm, out_hbm, tab_v, row_v, col_v, o0_v, o1_v):
    c = lax.axis_index("c")
    s = lax.axis_index("s")
    wid = c * NS + s
    pltpu.sync_copy(tab_hbm, tab_v)
    pltpu.sync_copy(row_hbm.at[pl.ds(wid * EPT, EPT)], row_v)
    pltpu.sync_copy(col_hbm.at[pl.ds(wid * EPT, EPT)], col_v)

    def body(i, carry):
        r16 = row_v[pl.ds(i * 16, 16)]
        c16 = col_v[pl.ds(i * 16, 16)]
        o0_v[pl.ds(i * 16, 16)] = (
            plsc.load_gather(tab_v, [r16])
            + plsc.load_gather(tab_v, [c16 + 2 * NPAD])
        )
        o1_v[pl.ds(i * 16, 16)] = (
            plsc.load_gather(tab_v, [r16 + NPAD])
            + plsc.load_gather(tab_v, [c16 + 3 * NPAD])
        )
        return carry

    lax.fori_loop(0, EPT // 16, body, 0)
    pltpu.sync_copy(o0_v, out_hbm.at[0].at[pl.ds(wid * EPT, EPT)])
    pltpu.sync_copy(o1_v, out_hbm.at[1].at[pl.ds(wid * EPT, EPT)])


def kernel(x, edge_index, W_self, W_neigh, b_core, W_node, b_node, W_edge, b_edge):
    row = edge_index[0].astype(jnp.int32)
    col = edge_index[1].astype(jnp.int32)
    xs = jnp.stack([x[:, :DH], x[:, DH:]])
    partials = _segment_sum_sc(
        xs, col.reshape(NS, NCHUNK, CHUNK), row.reshape(NS, NCHUNK, CHUNK)
    )
    w8 = (
        jnp.zeros((D, 8), jnp.float32)
        .at[:, 0:2].set(W_node)
        .at[:, 2:4].set(W_edge[:D])
        .at[:, 4:6].set(W_edge[D:])
    )
    b8 = (
        jnp.zeros((8,), jnp.float32)
        .at[0:2].set(b_node)
        .at[2:4].set(b_edge)
    )
    xp = jnp.pad(x, ((0, NPAD - N_NODES), (0, 0)))
    nlT, tab4 = _dense_tc(
        xp, partials, W_self, W_neigh, b_core.reshape(1, D), w8, b8.reshape(1, 8)
    )
    node_logits = nlT[:, :N_NODES].T
    edge_logits = _edge_logits_sc(tab4.reshape(-1), row, col).T
    return (node_logits, edge_logits)


# speedup vs baseline: 2.5170x; 1.0452x over previous
import functools

import jax
import jax.numpy as jnp
from jax import lax
from jax.experimental import pallas as pl
from jax.experimental.pallas import tpu as pltpu
from jax.experimental.pallas import tpu_sc as plsc

N_NODES = 10000
N_EDGES = 320000
D = 128
DH = D // 2
NC = 2
NS = 16
NW = NC * NS
EPT = N_EDGES // NW
EPT1 = N_EDGES // NS
CHUNK = 125
NCHUNK = EPT1 // CHUNK
RING = 4
NPAD = 10240
RPT = NPAD // NS

_mesh = plsc.VectorSubcoreMesh(
    core_axis_name="c", subcore_axis_name="s", num_cores=NC, num_subcores=NS
)


@functools.partial(
    pl.kernel,
    out_type=jax.ShapeDtypeStruct((NC, NPAD, DH), jnp.float32),
    mesh=_mesh,
    scratch_types=[
        pltpu.VMEM((NCHUNK, CHUNK), jnp.int32),
        pltpu.VMEM((NCHUNK, CHUNK), jnp.int32),
        pltpu.VMEM((RING, CHUNK, DH), jnp.float32),
        pltpu.VMEM((16, DH), jnp.float32),
        pltpu.VMEM_SHARED((NPAD, DH), jnp.float32),
        pltpu.SemaphoreType.DMA,
        pltpu.SemaphoreType.DMA,
        pltpu.SemaphoreType.DMA,
        pltpu.SemaphoreType.DMA,
    ],
    compiler_params=pltpu.CompilerParams(use_tc_tiling_on_sc=False),
)
def _segment_sum_sc(xs_hbm, ei_hbm, out_hbm,
                    col_v, row_v, bufs, zbuf, acc, sem0, sem1, sem2, sem3):
    c = lax.axis_index("c")
    s = lax.axis_index("s")

    zero16 = jnp.zeros((16,), jnp.float32)
    for i in range(16):
        for k in range(DH // 16):
            zbuf[i, pl.ds(k * 16, 16)] = zero16
    base_row = s * RPT
    for k in range(RPT // 16):
        pltpu.sync_copy(zbuf, acc.at[pl.ds(base_row + k * 16, 16)])
    plsc.subcore_barrier()

    pltpu.sync_copy(ei_hbm.at[1].at[s], col_v)
    pltpu.sync_copy(ei_hbm.at[0].at[s], row_v)

    x_hbm = xs_hbm.at[c]
    sems = (sem0, sem1, sem2, sem3)

    for b in range(RING - 1):
        pltpu.async_copy(x_hbm.at[col_v.at[b]], bufs.at[b], sems[b])

    def body(g, carry):
        for b in range(RING):
            j = g * RING + b

            @pl.when(j + RING - 1 < NCHUNK)
            def _():
                pltpu.async_copy(
                    x_hbm.at[col_v.at[j + RING - 1]],
                    bufs.at[(b + RING - 1) % RING],
                    sems[(b + RING - 1) % RING],
                )

            @pl.when(j < NCHUNK)
            def _():
                pltpu.make_async_copy(
                    x_hbm.at[col_v.at[j]], bufs.at[b], sems[b]
                ).wait()
                pltpu.sync_copy(bufs.at[b], acc.at[row_v.at[j]], add=True)

        return carry

    lax.fori_loop(0, (NCHUNK + RING - 1) // RING, body, 0)
    plsc.subcore_barrier()

    for k in range(RPT // 128):
        r0 = base_row + k * 128
        pltpu.sync_copy(acc.at[pl.ds(r0, 128)], out_hbm.at[c].at[pl.ds(r0, 128)])


def _dense_tc_body(x_ref, p_ref, ws_ref, wn_ref, bc_ref, w8_ref, b8_ref,
                   nl_ref, tab_ref):
    wn = wn_ref[...]
    h = jnp.maximum(
        jnp.dot(x_ref[...], ws_ref[...], preferred_element_type=jnp.float32)
        + jnp.dot(p_ref[0], wn[:DH, :], preferred_element_type=jnp.float32)
        + jnp.dot(p_ref[1], wn[DH:, :], preferred_element_type=jnp.float32)
        + bc_ref[...],
        0.0,
    )
    out8 = jnp.dot(h, w8_ref[...], preferred_element_type=jnp.float32) + b8_ref[...]
    nl_ref[...] = out8[:, 0:2].T
    tab_ref[...] = out8[:, 2:6].T


_BN = 1280


def _dense_tc(xp, partials, W_self, W_neigh, bc, w8, b8):
    return pl.pallas_call(
        _dense_tc_body,
        grid=(NPAD // _BN,),
        in_specs=[
            pl.BlockSpec((_BN, D), lambda i: (i, 0)),
            pl.BlockSpec((NC, _BN, DH), lambda i: (0, i, 0)),
            pl.BlockSpec((D, D), lambda i: (0, 0)),
            pl.BlockSpec((D, D), lambda i: (0, 0)),
            pl.BlockSpec((1, D), lambda i: (0, 0)),
            pl.BlockSpec((D, 8), lambda i: (0, 0)),
            pl.BlockSpec((1, 8), lambda i: (0, 0)),
        ],
        out_specs=[
            pl.BlockSpec((2, _BN), lambda i: (0, i)),
            pl.BlockSpec((4, _BN), lambda i: (0, i)),
        ],
        out_shape=[
            jax.ShapeDtypeStruct((2, NPAD), jnp.float32),
            jax.ShapeDtypeStruct((4, NPAD), jnp.float32),
        ],
    )(xp, partials, W_self, W_neigh, bc, w8, b8)


@functools.partial(
    pl.kernel,
    out_type=jax.ShapeDtypeStruct((2, N_EDGES), jnp.float32),
    mesh=_mesh,
    scratch_types=[
        pltpu.VMEM((NPAD * 4,), jnp.float32),
        pltpu.VMEM((EPT,), jnp.int32),
        pltpu.VMEM((EPT,), jnp.int32),
        pltpu.VMEM((EPT,), jnp.float32),
        pltpu.VMEM((EPT,), jnp.float32),
    ],
    compiler_params=pltpu.CompilerParams(
        needs_layout_passes=False, use_tc_tiling_on_sc=False
    ),
)
def _edge_logits_sc(tab_hbm, row_hbm, col_hbm, out_hbm, tab_v, row_v, col_v, o0_v, o1_v):
    c = lax.axis_index("c")
    s = lax.axis_index("s")
    wid = c * NS + s
    pltpu.sync_copy(tab_hbm, tab_v)
    pltpu.sync_copy(row_hbm.at[pl.ds(wid * EPT, EPT)], row_v)
    pltpu.sync_copy(col_hbm.at[pl.ds(wid * EPT, EPT)], col_v)

    def one(i):
        r16 = row_v[pl.ds(i * 16, 16)]
        c16 = col_v[pl.ds(i * 16, 16)]
        o0_v[pl.ds(i * 16, 16)] = (
            plsc.load_gather(tab_v, [r16])
            + plsc.load_gather(tab_v, [c16 + 2 * NPAD])
        )
        o1_v[pl.ds(i * 16, 16)] = (
            plsc.load_gather(tab_v, [r16 + NPAD])
            + plsc.load_gather(tab_v, [c16 + 3 * NPAD])
        )

    def body(g, carry):
        one(g * 2)
        one(g * 2 + 1)
        return carry

    lax.fori_loop(0, EPT // 32, body, 0)
    one(EPT // 16 - 1)
    pltpu.sync_copy(o0_v, out_hbm.at[0].at[pl.ds(wid * EPT, EPT)])
    pltpu.sync_copy(o1_v, out_hbm.at[1].at[pl.ds(wid * EPT, EPT)])


def kernel(x, edge_index, W_self, W_neigh, b_core, W_node, b_node, W_edge, b_edge):
    ei = edge_index.astype(jnp.int32)
    row = ei[0]
    col = ei[1]
    xs = jnp.stack([x[:, :DH], x[:, DH:]])
    partials = _segment_sum_sc(xs, ei.reshape(2, NS, NCHUNK, CHUNK))
    w8 = (
        jnp.zeros((D, 8), jnp.float32)
        .at[:, 0:2].set(W_node)
        .at[:, 2:4].set(W_edge[:D])
        .at[:, 4:6].set(W_edge[D:])
    )
    b8 = (
        jnp.zeros((8,), jnp.float32)
        .at[0:2].set(b_node)
        .at[2:4].set(b_edge)
    )
    xp = jnp.pad(x, ((0, NPAD - N_NODES), (0, 0)))
    nlT, tab4 = _dense_tc(
        xp, partials, W_self, W_neigh, b_core.reshape(1, D), w8, b8.reshape(1, 8)
    )
    node_logits = nlT[:, :N_NODES].T
    edge_logits = _edge_logits_sc(tab4.reshape(-1), row, col).T
    return (node_logits, edge_logits)


# speedup vs baseline: 2.6057x; 1.0352x over previous
import functools

import jax
import jax.numpy as jnp
from jax import lax
from jax.experimental import pallas as pl
from jax.experimental.pallas import tpu as pltpu
from jax.experimental.pallas import tpu_sc as plsc

N_NODES = 10000
N_EDGES = 320000
D = 128
DH = D // 2
NC = 2
NS = 16
NW = NC * NS
EPT = N_EDGES // NW
EPT1 = N_EDGES // NS
CHUNK = 160
NCHUNK = EPT1 // CHUNK
RING = 4
NPAD = 10240
RPT = NPAD // NS

_mesh = plsc.VectorSubcoreMesh(
    core_axis_name="c", subcore_axis_name="s", num_cores=NC, num_subcores=NS
)


@functools.partial(
    pl.kernel,
    out_type=jax.ShapeDtypeStruct((NC, NPAD, DH), jnp.float32),
    mesh=_mesh,
    scratch_types=[
        pltpu.VMEM((EPT1,), jnp.int32),
        pltpu.VMEM((EPT1,), jnp.int32),
        pltpu.VMEM((RING, CHUNK, DH), jnp.float32),
        pltpu.VMEM((16, DH), jnp.float32),
        pltpu.VMEM_SHARED((NPAD, DH), jnp.float32),
        pltpu.SemaphoreType.DMA,
        pltpu.SemaphoreType.DMA,
        pltpu.SemaphoreType.DMA,
        pltpu.SemaphoreType.DMA,
    ],
    compiler_params=pltpu.CompilerParams(use_tc_tiling_on_sc=False),
)
def _segment_sum_sc(xs_hbm, ei_hbm, out_hbm,
                    col_v, row_v, bufs, zbuf, acc, sem0, sem1, sem2, sem3):
    c = lax.axis_index("c")
    s = lax.axis_index("s")

    zero16 = jnp.zeros((16,), jnp.float32)
    for i in range(16):
        for k in range(DH // 16):
            zbuf[i, pl.ds(k * 16, 16)] = zero16
    base_row = s * RPT
    for k in range(RPT // 16):
        pltpu.sync_copy(zbuf, acc.at[pl.ds(base_row + k * 16, 16)])
    plsc.subcore_barrier()

    pltpu.sync_copy(ei_hbm.at[1].at[pl.ds(s * EPT1, EPT1)], col_v)
    pltpu.sync_copy(ei_hbm.at[0].at[pl.ds(s * EPT1, EPT1)], row_v)

    x_hbm = xs_hbm.at[c]
    sems = (sem0, sem1, sem2, sem3)

    def cidx(j):
        return col_v.at[pl.ds(j * CHUNK, CHUNK)]

    def ridx(j):
        return row_v.at[pl.ds(j * CHUNK, CHUNK)]

    for b in range(RING - 1):
        pltpu.async_copy(x_hbm.at[cidx(b)], bufs.at[b], sems[b])

    def body(g, carry):
        for b in range(RING):
            j = g * RING + b

            @pl.when(j + RING - 1 < NCHUNK)
            def _():
                pltpu.async_copy(
                    x_hbm.at[cidx(j + RING - 1)],
                    bufs.at[(b + RING - 1) % RING],
                    sems[(b + RING - 1) % RING],
                )

            @pl.when(j < NCHUNK)
            def _():
                pltpu.make_async_copy(
                    x_hbm.at[cidx(j)], bufs.at[b], sems[b]
                ).wait()
                pltpu.sync_copy(bufs.at[b], acc.at[ridx(j)], add=True)

        return carry

    lax.fori_loop(0, (NCHUNK + RING - 1) // RING, body, 0)
    plsc.subcore_barrier()

    for k in range(RPT // 128):
        r0 = base_row + k * 128
        pltpu.sync_copy(acc.at[pl.ds(r0, 128)], out_hbm.at[c].at[pl.ds(r0, 128)])


def _dense_tc_body(x_ref, p_ref, ws_ref, wn_ref, bc_ref, w8_ref, b8_ref,
                   nl_ref, tab_ref):
    wn = wn_ref[...]
    h = jnp.maximum(
        jnp.dot(x_ref[...], ws_ref[...], preferred_element_type=jnp.float32)
        + jnp.dot(p_ref[0], wn[:DH, :], preferred_element_type=jnp.float32)
        + jnp.dot(p_ref[1], wn[DH:, :], preferred_element_type=jnp.float32)
        + bc_ref[...],
        0.0,
    )
    out8 = jnp.dot(h, w8_ref[...], preferred_element_type=jnp.float32) + b8_ref[...]
    nl_ref[...] = out8[:, 0:2].T
    tab_ref[...] = out8[:, 2:6].T


_BN = 1280


def _dense_tc(xp, partials, W_self, W_neigh, bc, w8, b8):
    return pl.pallas_call(
        _dense_tc_body,
        grid=(NPAD // _BN,),
        in_specs=[
            pl.BlockSpec((_BN, D), lambda i: (i, 0)),
            pl.BlockSpec((NC, _BN, DH), lambda i: (0, i, 0)),
            pl.BlockSpec((D, D), lambda i: (0, 0)),
            pl.BlockSpec((D, D), lambda i: (0, 0)),
            pl.BlockSpec((1, D), lambda i: (0, 0)),
            pl.BlockSpec((D, 8), lambda i: (0, 0)),
            pl.BlockSpec((1, 8), lambda i: (0, 0)),
        ],
        out_specs=[
            pl.BlockSpec((2, _BN), lambda i: (0, i)),
            pl.BlockSpec((4, _BN), lambda i: (0, i)),
        ],
        out_shape=[
            jax.ShapeDtypeStruct((2, NPAD), jnp.float32),
            jax.ShapeDtypeStruct((4, NPAD), jnp.float32),
        ],
    )(xp, partials, W_self, W_neigh, bc, w8, b8)


@functools.partial(
    pl.kernel,
    out_type=jax.ShapeDtypeStruct((2, N_EDGES), jnp.float32),
    mesh=_mesh,
    scratch_types=[
        pltpu.VMEM((NPAD * 4,), jnp.float32),
        pltpu.VMEM((EPT,), jnp.int32),
        pltpu.VMEM((EPT,), jnp.int32),
        pltpu.VMEM((EPT,), jnp.float32),
        pltpu.VMEM((EPT,), jnp.float32),
    ],
    compiler_params=pltpu.CompilerParams(
        needs_layout_passes=False, use_tc_tiling_on_sc=False
    ),
)
def _edge_logits_sc(tab_hbm, ei_hbm, out_hbm, tab_v, row_v, col_v, o0_v, o1_v):
    c = lax.axis_index("c")
    s = lax.axis_index("s")
    wid = c * NS + s
    pltpu.sync_copy(tab_hbm, tab_v)
    pltpu.sync_copy(ei_hbm.at[0].at[pl.ds(wid * EPT, EPT)], row_v)
    pltpu.sync_copy(ei_hbm.at[1].at[pl.ds(wid * EPT, EPT)], col_v)

    def one(i):
        r16 = row_v[pl.ds(i * 16, 16)]
        c16 = col_v[pl.ds(i * 16, 16)]
        o0_v[pl.ds(i * 16, 16)] = (
            plsc.load_gather(tab_v, [r16])
            + plsc.load_gather(tab_v, [c16 + 2 * NPAD])
        )
        o1_v[pl.ds(i * 16, 16)] = (
            plsc.load_gather(tab_v, [r16 + NPAD])
            + plsc.load_gather(tab_v, [c16 + 3 * NPAD])
        )

    def body(g, carry):
        one(g * 2)
        one(g * 2 + 1)
        return carry

    lax.fori_loop(0, EPT // 32, body, 0)
    one(EPT // 16 - 1)
    pltpu.sync_copy(o0_v, out_hbm.at[0].at[pl.ds(wid * EPT, EPT)])
    pltpu.sync_copy(o1_v, out_hbm.at[1].at[pl.ds(wid * EPT, EPT)])


def kernel(x, edge_index, W_self, W_neigh, b_core, W_node, b_node, W_edge, b_edge):
    ei = edge_index.astype(jnp.int32)
    xs = jnp.stack([x[:, :DH], x[:, DH:]])
    partials = _segment_sum_sc(xs, ei)
    w8 = (
        jnp.zeros((D, 8), jnp.float32)
        .at[:, 0:2].set(W_node)
        .at[:, 2:4].set(W_edge[:D])
        .at[:, 4:6].set(W_edge[D:])
    )
    b8 = (
        jnp.zeros((8,), jnp.float32)
        .at[0:2].set(b_node)
        .at[2:4].set(b_edge)
    )
    xp = jnp.pad(x, ((0, NPAD - N_NODES), (0, 0)))
    nlT, tab4 = _dense_tc(
        xp, partials, W_self, W_neigh, b_core.reshape(1, D), w8, b8.reshape(1, 8)
    )
    node_logits = nlT[:, :N_NODES].T
    edge_logits = _edge_logits_sc(tab4.reshape(-1), ei).T
    return (node_logits, edge_logits)
